# baseline (device time: 241534 ns/iter reference)
import jax
import jax.numpy as jnp
from jax import lax
from jax.experimental import pallas as pl
from jax.experimental.pallas import tpu as pltpu

M = 4096
N = 8192
K = 4096
HALF = M // 2
NDEV = 8
SW = N // NDEV
HW = SW // 2

_BM = 256
_BK = 1024
_NB = HALF // _BM
_NK = K // _BK
_NHOP = NDEV - 1
_HOP_STEPS = 4

_LAST_STEP = 2 * _NB * _NK - 1
_SCHED: dict[int, list[tuple[int, int]]] = {}
_TAIL: list[tuple[int, int, int]] = []
for _b in range(_NB):
    for _h in range(_NHOP):
        _s = (2 * _b + 1) * _NK + _NK - 1 + _HOP_STEPS * _h
        if _s < _LAST_STEP:
            _SCHED.setdefault(_s, []).append((_b, _h))
        else:
            _TAIL.append((_s, _b, _h))
_TAIL.sort()


def _h1_pos(y, z):
    return 2 * z + (y + z) % 2


def _h1_yz(p):
    z = p // 2
    y = (p % 2 + z) % 2
    return y, z


def _h2_pos(y, z):
    q0 = z + 2 * (z // 2)
    q1 = jnp.where(z == 0, 7, jnp.where(z == 3, 6, z + 1))
    return jnp.where(y == 0, q0, q1)


def _h2_yz(q):
    y = (q // 2) % 2
    z = ((q + 1) // 2) % 4
    return y, z


def _body(s_ref, x_ref, dy_ref, out_ref, acc, sblk, rbuf, fvm,
          msend, mrecv, csem, a_send, a_recv, b_send, b_recv):
    i = pl.program_id(0)
    k = pl.program_id(1)
    step = i * _NK + k
    g = s_ref[0]
    y = lax.axis_index("y")
    z = lax.axis_index("z")
    p = _h1_pos(y, z)
    q = _h2_pos(y, z)
    ay, az = _h1_yz((p + 1) % NDEV)
    by, bz = _h2_yz((q + 1) % NDEV)

    def a_col(h):
        return ((p - h) % NDEV) * SW

    def b_col(h):
        oby, obz = _h2_yz((q - h) % NDEV)
        return _h1_pos(oby, obz) * SW + HW

    def ring_desc(ring, b, h):
        col = a_col(h) if ring == 0 else b_col(h)
        dst = out_ref.at[pl.ds(b * _BM, _BM), pl.ds(col, HW)]
        if h == 0:
            src = fvm.at[b, :, pl.ds(0 if ring == 0 else HW, HW)]
        else:
            src = out_ref.at[pl.ds(b * _BM, _BM), pl.ds(col, HW)]
        ss, rs = (a_send, a_recv) if ring == 0 else (b_send, b_recv)
        dev = (g, ay, az) if ring == 0 else (g, by, bz)
        return pltpu.make_async_remote_copy(
            src_ref=src, dst_ref=dst,
            send_sem=ss.at[b, h], recv_sem=rs.at[b, h],
            device_id=dev, device_id_type=pl.DeviceIdType.MESH,
        )

    def ring_op(b, h):
        for ring in (0, 1):
            if h > 0:
                col = a_col(h) if ring == 0 else b_col(h)
                dst = out_ref.at[pl.ds(b * _BM, _BM), pl.ds(col, HW)]
                ss, rs = (a_send, a_recv) if ring == 0 else (b_send, b_recv)
                dev = (g, ay, az) if ring == 0 else (g, by, bz)
                pltpu.make_async_remote_copy(
                    src_ref=dst, dst_ref=dst,
                    send_sem=ss.at[b, h - 1], recv_sem=rs.at[b, h - 1],
                    device_id=dev, device_id_type=pl.DeviceIdType.MESH,
                ).wait_recv()
            ring_desc(ring, b, h).start()

    def mirror_desc(b):
        return pltpu.make_async_remote_copy(
            src_ref=sblk.at[b], dst_ref=rbuf.at[b],
            send_sem=msend.at[b], recv_sem=mrecv.at[b],
            device_id=(1 - g, y, z), device_id_type=pl.DeviceIdType.MESH,
        )

    def out_copy(b):
        return pltpu.make_async_copy(
            fvm.at[b], out_ref.at[pl.ds(b * _BM, _BM), pl.ds(p * SW, SW)],
            csem.at[b],
        )

    @pl.when(k == 0)
    def _():
        acc[...] = jnp.zeros_like(acc)

    xb = x_ref[...].astype(jnp.bfloat16)
    db = dy_ref[...].astype(jnp.bfloat16)
    acc[...] += lax.dot_general(
        xb, db, (((0,), (0,)), ((), ())), preferred_element_type=jnp.float32
    )

    @pl.when(k == _NK - 1)
    def _():
        for b in range(_NB):
            @pl.when(i == 2 * b)
            def _():
                sblk[b] = acc[...].astype(jnp.bfloat16)
                mirror_desc(b).start()

            @pl.when(i == 2 * b + 1)
            def _():
                mirror_desc(b).wait_recv()
                fvm[b] = (acc[...] + rbuf[b].astype(jnp.float32)).astype(
                    jnp.bfloat16
                )
                out_copy(b).start()

    for s, ops in sorted(_SCHED.items()):
        @pl.when(step == s)
        def _(ops=ops):
            for b, h in ops:
                ring_op(b, h)

    @pl.when(step == _LAST_STEP)
    def _():
        for _, b, h in _TAIL:
            ring_op(b, h)

        for b in range(_NB):
            for ring in (0, 1):
                col = a_col(_NHOP) if ring == 0 else b_col(_NHOP)
                dst = out_ref.at[pl.ds(b * _BM, _BM), pl.ds(col, HW)]
                ss, rs = (a_send, a_recv) if ring == 0 else (b_send, b_recv)
                dev = (g, ay, az) if ring == 0 else (g, by, bz)
                pltpu.make_async_remote_copy(
                    src_ref=dst, dst_ref=dst,
                    send_sem=ss.at[b, _NHOP - 1], recv_sem=rs.at[b, _NHOP - 1],
                    device_id=dev, device_id_type=pl.DeviceIdType.MESH,
                ).wait_recv()
        for b in range(_NB):
            mirror_desc(b).wait_send()
            out_copy(b).wait()
            for h in range(_NHOP):
                for ring in (0, 1):
                    ring_desc(ring, b, h).wait_send()


def kernel(x, dy):
    g = lax.axis_index("x")
    y = lax.axis_index("y")
    z = lax.axis_index("z")
    p = _h1_pos(y, z)
    scalars = jnp.stack([g, p]).astype(jnp.int32)

    grid_spec = pltpu.PrefetchScalarGridSpec(
        num_scalar_prefetch=1,
        grid=(2 * _NB, _NK),
        in_specs=[
            pl.BlockSpec(
                (_BK, _BM),
                lambda i, k, s: (
                    k,
                    jnp.where(
                        i % 2 == 0,
                        (1 - s[0]) * _NB + i // 2,
                        s[0] * _NB + i // 2,
                    ),
                ),
            ),
            pl.BlockSpec((_BK, SW), lambda i, k, s: (k, s[1])),
        ],
        out_specs=pl.BlockSpec(memory_space=pltpu.MemorySpace.HBM),
        scratch_shapes=[
            pltpu.VMEM((_BM, SW), jnp.float32),
            pltpu.VMEM((_NB, _BM, SW), jnp.bfloat16),
            pltpu.VMEM((_NB, _BM, SW), jnp.bfloat16),
            pltpu.VMEM((_NB, _BM, SW), jnp.bfloat16),
            pltpu.SemaphoreType.DMA((_NB,)),
            pltpu.SemaphoreType.DMA((_NB,)),
            pltpu.SemaphoreType.DMA((_NB,)),
            pltpu.SemaphoreType.DMA((_NB, _NHOP)),
            pltpu.SemaphoreType.DMA((_NB, _NHOP)),
            pltpu.SemaphoreType.DMA((_NB, _NHOP)),
            pltpu.SemaphoreType.DMA((_NB, _NHOP)),
        ],
    )
    return pl.pallas_call(
        _body,
        grid_spec=grid_spec,
        out_shape=jax.ShapeDtypeStruct((HALF, N), jnp.bfloat16),
        compiler_params=pltpu.CompilerParams(
            dimension_semantics=("arbitrary", "arbitrary"),
        ),
    )(scalars, x, dy)


# device time: 226993 ns/iter; 1.0641x vs baseline; 1.0641x over previous
import jax
import jax.numpy as jnp
from jax import lax
from jax.experimental import pallas as pl
from jax.experimental.pallas import tpu as pltpu

M = 4096
N = 8192
K = 4096
HALF = M // 2
NDEV = 8
SW = N // NDEV
HW = SW // 2

_BM = 512
_BK = 1024
_NB = HALF // _BM
_NK = K // _BK
_NHOP = NDEV - 1

_LAST_STEP = 2 * _NB * _NK - 1
_SCHED: dict[int, list[tuple[int, int]]] = {}
_TAIL: list[tuple[int, int, int]] = []
for _b in range(_NB):
    for _h in range(_NHOP):
        _s = 8 * _b + 7 + 3 * _h
        if _s < _LAST_STEP:
            _SCHED.setdefault(_s, []).append((_b, _h))
        else:
            _TAIL.append((_s, _b, _h))
_TAIL.sort()


def _h1_pos(y, z):
    return 2 * z + (y + z) % 2


def _h1_yz(p):
    z = p // 2
    y = (p % 2 + z) % 2
    return y, z


def _h2_pos(y, z):
    q0 = z + 2 * (z // 2)
    q1 = jnp.where(z == 0, 7, jnp.where(z == 3, 6, z + 1))
    return jnp.where(y == 0, q0, q1)


def _h2_yz(q):
    y = (q // 2) % 2
    z = ((q + 1) // 2) % 4
    return y, z


def _body(s_ref, x_ref, dy_ref, out_ref, acc, sblk, rbuf, fvm,
          msend, mrecv, csem, a_send, a_recv, b_send, b_recv):
    i = pl.program_id(0)
    k = pl.program_id(1)
    step = i * _NK + k
    g = s_ref[0]
    y = lax.axis_index("y")
    z = lax.axis_index("z")
    p = _h1_pos(y, z)
    q = _h2_pos(y, z)
    ay, az = _h1_yz((p + 1) % NDEV)
    by, bz = _h2_yz((q + 1) % NDEV)

    def a_col(h):
        return ((p - h) % NDEV) * SW

    def b_col(h):
        oby, obz = _h2_yz((q - h) % NDEV)
        return _h1_pos(oby, obz) * SW + HW

    def ring_desc(ring, b, h):
        col = a_col(h) if ring == 0 else b_col(h)
        dst = out_ref.at[pl.ds(b * _BM, _BM), pl.ds(col, HW)]
        if h == 0:
            src = fvm.at[b, :, pl.ds(0 if ring == 0 else HW, HW)]
        else:
            src = out_ref.at[pl.ds(b * _BM, _BM), pl.ds(col, HW)]
        ss, rs = (a_send, a_recv) if ring == 0 else (b_send, b_recv)
        dev = (g, ay, az) if ring == 0 else (g, by, bz)
        return pltpu.make_async_remote_copy(
            src_ref=src, dst_ref=dst,
            send_sem=ss.at[b, h], recv_sem=rs.at[b, h],
            device_id=dev, device_id_type=pl.DeviceIdType.MESH,
        )

    def ring_op(b, h):
        for ring in (0, 1):
            if h > 0:
                col = a_col(h) if ring == 0 else b_col(h)
                dst = out_ref.at[pl.ds(b * _BM, _BM), pl.ds(col, HW)]
                ss, rs = (a_send, a_recv) if ring == 0 else (b_send, b_recv)
                dev = (g, ay, az) if ring == 0 else (g, by, bz)
                pltpu.make_async_remote_copy(
                    src_ref=dst, dst_ref=dst,
                    send_sem=ss.at[b, h - 1], recv_sem=rs.at[b, h - 1],
                    device_id=dev, device_id_type=pl.DeviceIdType.MESH,
                ).wait_recv()
            ring_desc(ring, b, h).start()

    def mirror_desc(b):
        return pltpu.make_async_remote_copy(
            src_ref=sblk.at[b], dst_ref=rbuf.at[b],
            send_sem=msend.at[b], recv_sem=mrecv.at[b],
            device_id=(1 - g, y, z), device_id_type=pl.DeviceIdType.MESH,
        )

    def out_copy(b):
        return pltpu.make_async_copy(
            fvm.at[b], out_ref.at[pl.ds(b * _BM, _BM), pl.ds(p * SW, SW)],
            csem.at[b],
        )

    @pl.when(k == 0)
    def _():
        acc[...] = jnp.zeros_like(acc)

    xb = x_ref[...].astype(jnp.bfloat16)
    db = dy_ref[...].astype(jnp.bfloat16)
    acc[...] += lax.dot_general(
        xb, db, (((0,), (0,)), ((), ())), preferred_element_type=jnp.float32
    )

    @pl.when(k == _NK - 1)
    def _():
        for b in range(_NB):
            @pl.when(i == 2 * b)
            def _():
                sblk[b] = acc[...].astype(jnp.bfloat16)
                mirror_desc(b).start()

            @pl.when(i == 2 * b + 1)
            def _():
                mirror_desc(b).wait_recv()
                fvm[b] = (acc[...] + rbuf[b].astype(jnp.float32)).astype(
                    jnp.bfloat16
                )
                out_copy(b).start()

    for s, ops in sorted(_SCHED.items()):
        @pl.when(step == s)
        def _(ops=ops):
            for b, h in ops:
                ring_op(b, h)

    @pl.when(step == _LAST_STEP)
    def _():
        for _, b, h in _TAIL:
            ring_op(b, h)

        for b in range(_NB):
            for ring in (0, 1):
                col = a_col(_NHOP) if ring == 0 else b_col(_NHOP)
                dst = out_ref.at[pl.ds(b * _BM, _BM), pl.ds(col, HW)]
                ss, rs = (a_send, a_recv) if ring == 0 else (b_send, b_recv)
                dev = (g, ay, az) if ring == 0 else (g, by, bz)
                pltpu.make_async_remote_copy(
                    src_ref=dst, dst_ref=dst,
                    send_sem=ss.at[b, _NHOP - 1], recv_sem=rs.at[b, _NHOP - 1],
                    device_id=dev, device_id_type=pl.DeviceIdType.MESH,
                ).wait_recv()
        for b in range(_NB):
            mirror_desc(b).wait_send()
            out_copy(b).wait()
            for h in range(_NHOP):
                for ring in (0, 1):
                    ring_desc(ring, b, h).wait_send()


def kernel(x, dy):
    g = lax.axis_index("x")
    y = lax.axis_index("y")
    z = lax.axis_index("z")
    p = _h1_pos(y, z)
    scalars = jnp.stack([g, p]).astype(jnp.int32)

    grid_spec = pltpu.PrefetchScalarGridSpec(
        num_scalar_prefetch=1,
        grid=(2 * _NB, _NK),
        in_specs=[
            pl.BlockSpec(
                (_BK, _BM),
                lambda i, k, s: (
                    k,
                    jnp.where(
                        i % 2 == 0,
                        (1 - s[0]) * _NB + i // 2,
                        s[0] * _NB + i // 2,
                    ),
                ),
            ),
            pl.BlockSpec((_BK, SW), lambda i, k, s: (k, s[1])),
        ],
        out_specs=pl.BlockSpec(memory_space=pltpu.MemorySpace.HBM),
        scratch_shapes=[
            pltpu.VMEM((_BM, SW), jnp.float32),
            pltpu.VMEM((_NB, _BM, SW), jnp.bfloat16),
            pltpu.VMEM((_NB, _BM, SW), jnp.bfloat16),
            pltpu.VMEM((_NB, _BM, SW), jnp.bfloat16),
            pltpu.SemaphoreType.DMA((_NB,)),
            pltpu.SemaphoreType.DMA((_NB,)),
            pltpu.SemaphoreType.DMA((_NB,)),
            pltpu.SemaphoreType.DMA((_NB, _NHOP)),
            pltpu.SemaphoreType.DMA((_NB, _NHOP)),
            pltpu.SemaphoreType.DMA((_NB, _NHOP)),
            pltpu.SemaphoreType.DMA((_NB, _NHOP)),
        ],
    )
    return pl.pallas_call(
        _body,
        grid_spec=grid_spec,
        out_shape=jax.ShapeDtypeStruct((HALF, N), jnp.bfloat16),
        compiler_params=pltpu.CompilerParams(
            dimension_semantics=("arbitrary", "arbitrary"),
        ),
    )(scalars, x, dy)


# device time: 226779 ns/iter; 1.0651x vs baseline; 1.0009x over previous
import jax
import jax.numpy as jnp
from jax import lax
from jax.experimental import pallas as pl
from jax.experimental.pallas import tpu as pltpu

M = 4096
N = 8192
K = 4096
HALF = M // 2
NDEV = 8
SW = N // NDEV
HW = SW // 2

_BM = 512
_BK = 1024
_NB = HALF // _BM
_NK = K // _BK
_NHOP = NDEV - 1

_LAST_STEP = 2 * _NB * _NK - 1
_SCHED: dict[int, list[tuple[int, int]]] = {}
_TAIL: list[tuple[int, int, int]] = []
for _b in range(_NB):
    for _h in range(_NHOP):
        _s = 8 * _b + 7 + 3 * _h
        if _s < _LAST_STEP:
            _SCHED.setdefault(_s, []).append((_b, _h))
        else:
            _TAIL.append((_s, _b, _h))
_TAIL.sort()


def _h1_pos(y, z):
    return 2 * z + (y + z) % 2


def _h1_yz(p):
    z = p // 2
    y = (p % 2 + z) % 2
    return y, z


def _h2_pos(y, z):
    q0 = z + 2 * (z // 2)
    q1 = jnp.where(z == 0, 7, jnp.where(z == 3, 6, z + 1))
    return jnp.where(y == 0, q0, q1)


def _h2_yz(q):
    y = (q // 2) % 2
    z = ((q + 1) // 2) % 4
    return y, z


def _body(s_ref, x_ref, dy_ref, out_ref, acc, sblk, rbuf, fvm,
          msend, mrecv, csem, a_send, a_recv, b_send, b_recv):
    i = pl.program_id(0)
    k = pl.program_id(1)
    step = i * _NK + k
    g = s_ref[0]
    y = lax.axis_index("y")
    z = lax.axis_index("z")
    p = _h1_pos(y, z)
    q = _h2_pos(y, z)
    ay, az = _h1_yz((p + 1) % NDEV)
    by, bz = _h2_yz((q + 1) % NDEV)

    def a_col(h):
        return ((p - h) % NDEV) * SW

    def b_col(h):
        oby, obz = _h2_yz((q - h) % NDEV)
        return _h1_pos(oby, obz) * SW

    def ring_desc(ring, b, h):
        col = a_col(h) if ring == 0 else b_col(h)
        row = b * _BM + ring * (_BM // 2)
        dst = out_ref.at[pl.ds(row, _BM // 2), pl.ds(col, SW)]
        if h == 0:
            src = fvm.at[b, pl.ds(ring * (_BM // 2), _BM // 2), :]
        else:
            src = out_ref.at[pl.ds(row, _BM // 2), pl.ds(col, SW)]
        ss, rs = (a_send, a_recv) if ring == 0 else (b_send, b_recv)
        dev = (g, ay, az) if ring == 0 else (g, by, bz)
        return pltpu.make_async_remote_copy(
            src_ref=src, dst_ref=dst,
            send_sem=ss.at[b, h], recv_sem=rs.at[b, h],
            device_id=dev, device_id_type=pl.DeviceIdType.MESH,
        )

    def ring_op(b, h):
        for ring in (0, 1):
            if h > 0:
                col = a_col(h) if ring == 0 else b_col(h)
                row = b * _BM + ring * (_BM // 2)
                dst = out_ref.at[pl.ds(row, _BM // 2), pl.ds(col, SW)]
                ss, rs = (a_send, a_recv) if ring == 0 else (b_send, b_recv)
                dev = (g, ay, az) if ring == 0 else (g, by, bz)
                pltpu.make_async_remote_copy(
                    src_ref=dst, dst_ref=dst,
                    send_sem=ss.at[b, h - 1], recv_sem=rs.at[b, h - 1],
                    device_id=dev, device_id_type=pl.DeviceIdType.MESH,
                ).wait_recv()
            ring_desc(ring, b, h).start()

    def mirror_desc(b):
        return pltpu.make_async_remote_copy(
            src_ref=sblk.at[b], dst_ref=rbuf.at[b],
            send_sem=msend.at[b], recv_sem=mrecv.at[b],
            device_id=(1 - g, y, z), device_id_type=pl.DeviceIdType.MESH,
        )

    def out_copy(b):
        return pltpu.make_async_copy(
            fvm.at[b], out_ref.at[pl.ds(b * _BM, _BM), pl.ds(p * SW, SW)],
            csem.at[b],
        )

    @pl.when(k == 0)
    def _():
        acc[...] = jnp.zeros_like(acc)

    xb = x_ref[...].astype(jnp.bfloat16)
    db = dy_ref[...].astype(jnp.bfloat16)
    acc[...] += lax.dot_general(
        xb, db, (((0,), (0,)), ((), ())), preferred_element_type=jnp.float32
    )

    @pl.when(k == _NK - 1)
    def _():
        for b in range(_NB):
            @pl.when(i == 2 * b)
            def _():
                sblk[b] = acc[...].astype(jnp.bfloat16)
                mirror_desc(b).start()

            @pl.when(i == 2 * b + 1)
            def _():
                mirror_desc(b).wait_recv()
                fvm[b] = (acc[...] + rbuf[b].astype(jnp.float32)).astype(
                    jnp.bfloat16
                )
                out_copy(b).start()

    for s, ops in sorted(_SCHED.items()):
        @pl.when(step == s)
        def _(ops=ops):
            for b, h in ops:
                ring_op(b, h)

    @pl.when(step == _LAST_STEP)
    def _():
        for _, b, h in _TAIL:
            ring_op(b, h)

        for b in range(_NB):
            for ring in (0, 1):
                col = a_col(_NHOP) if ring == 0 else b_col(_NHOP)
                row = b * _BM + ring * (_BM // 2)
                dst = out_ref.at[pl.ds(row, _BM // 2), pl.ds(col, SW)]
                ss, rs = (a_send, a_recv) if ring == 0 else (b_send, b_recv)
                dev = (g, ay, az) if ring == 0 else (g, by, bz)
                pltpu.make_async_remote_copy(
                    src_ref=dst, dst_ref=dst,
                    send_sem=ss.at[b, _NHOP - 1], recv_sem=rs.at[b, _NHOP - 1],
                    device_id=dev, device_id_type=pl.DeviceIdType.MESH,
                ).wait_recv()
        for b in range(_NB):
            mirror_desc(b).wait_send()
            out_copy(b).wait()
            for h in range(_NHOP):
                for ring in (0, 1):
                    ring_desc(ring, b, h).wait_send()


def kernel(x, dy):
    g = lax.axis_index("x")
    y = lax.axis_index("y")
    z = lax.axis_index("z")
    p = _h1_pos(y, z)
    scalars = jnp.stack([g, p]).astype(jnp.int32)

    grid_spec = pltpu.PrefetchScalarGridSpec(
        num_scalar_prefetch=1,
        grid=(2 * _NB, _NK),
        in_specs=[
            pl.BlockSpec(
                (_BK, _BM),
                lambda i, k, s: (
                    k,
                    jnp.where(
                        i % 2 == 0,
                        (1 - s[0]) * _NB + i // 2,
                        s[0] * _NB + i // 2,
                    ),
                ),
            ),
            pl.BlockSpec((_BK, SW), lambda i, k, s: (k, s[1])),
        ],
        out_specs=pl.BlockSpec(memory_space=pltpu.MemorySpace.HBM),
        scratch_shapes=[
            pltpu.VMEM((_BM, SW), jnp.float32),
            pltpu.VMEM((_NB, _BM, SW), jnp.bfloat16),
            pltpu.VMEM((_NB, _BM, SW), jnp.bfloat16),
            pltpu.VMEM((_NB, _BM, SW), jnp.bfloat16),
            pltpu.SemaphoreType.DMA((_NB,)),
            pltpu.SemaphoreType.DMA((_NB,)),
            pltpu.SemaphoreType.DMA((_NB,)),
            pltpu.SemaphoreType.DMA((_NB, _NHOP)),
            pltpu.SemaphoreType.DMA((_NB, _NHOP)),
            pltpu.SemaphoreType.DMA((_NB, _NHOP)),
            pltpu.SemaphoreType.DMA((_NB, _NHOP)),
        ],
    )
    return pl.pallas_call(
        _body,
        grid_spec=grid_spec,
        out_shape=jax.ShapeDtypeStruct((HALF, N), jnp.bfloat16),
        compiler_params=pltpu.CompilerParams(
            dimension_semantics=("arbitrary", "arbitrary"),
        ),
    )(scalars, x, dy)
